# Initial kernel scaffold; baseline (speedup 1.0000x reference)
import jax
import jax.numpy as jnp
from jax.experimental import pallas as pl


def kernel(x, edge_index, Wl1, Wr1, b1, Wl2, Wr2, b2, Wl3, Wr3, b3):
    raise NotImplementedError("write your pallas kernel here")


# speedup vs baseline: 6.2080x; 6.2080x over previous
import functools

import jax
import jax.numpy as jnp
from jax import lax
from jax.experimental import pallas as pl
from jax.experimental.pallas import tpu as pltpu
from jax.experimental.pallas import tpu_sc as plsc

_N = 10000
_E = 320000
_CH = 128
_NW = 32
_ROWS = 1000
_F32 = jnp.float32


def _mesh():
    return plsc.VectorSubcoreMesh(core_axis_name="c", subcore_axis_name="s")


def _make_sc_agg(H):
    nchunks = _E // _CH
    base = nchunks // _NW
    extra = nchunks % _NW

    @functools.partial(
        pl.kernel,
        mesh=_mesh(),
        out_type=jax.ShapeDtypeStruct((2, _N, H), _F32),
        scratch_types=[
            pltpu.VMEM((_CH,), jnp.int32),
            pltpu.VMEM((_CH,), jnp.int32),
            pltpu.VMEM((_CH, H), _F32),
            pltpu.VMEM_SHARED((_N, H), _F32),
            pltpu.SemaphoreType.DMA,
        ],
    )
    def agg(hl_hbm, src_hbm, dst_hbm, z_hbm, out_hbm,
            src_v, dst_v, rows_v, acc_sh, sem):
        c = lax.axis_index("c")
        s = lax.axis_index("s")
        w = s * 2 + c

        @pl.when(s == 0)
        def _():
            pltpu.sync_copy(z_hbm, acc_sh)

        plsc.subcore_barrier()

        nch = base + jnp.where(w < extra, 1, 0)

        def chunk(i, carry):
            off = (i * _NW + w) * _CH
            pltpu.sync_copy(src_hbm.at[pl.ds(off, _CH)], src_v)
            pltpu.sync_copy(dst_hbm.at[pl.ds(off, _CH)], dst_v)
            pltpu.async_copy(hl_hbm.at[src_v], rows_v, sem).wait()
            pltpu.sync_copy(rows_v, acc_sh.at[dst_v], add=True)
            return carry

        lax.fori_loop(0, nch, chunk, 0)

        plsc.subcore_barrier()

        @pl.when(s == 0)
        def _():
            pltpu.sync_copy(acc_sh, out_hbm.at[c])

    return agg


def _make_sc_deg():
    nchunks = _E // _CH
    base = nchunks // _NW
    extra = nchunks % _NW

    @functools.partial(
        pl.kernel,
        mesh=_mesh(),
        out_type=jax.ShapeDtypeStruct((2, _N), _F32),
        scratch_types=[
            pltpu.VMEM((_CH,), jnp.int32),
            pltpu.VMEM((_CH,), _F32),
            pltpu.VMEM_SHARED((_N,), _F32),
            pltpu.SemaphoreType.DMA,
        ],
    )
    def deg(dst_hbm, zn_hbm, out_hbm, dst_v, ones_v, deg_sh, sem):
        c = lax.axis_index("c")
        s = lax.axis_index("s")
        w = s * 2 + c

        for j in range(_CH // 16):
            ones_v[pl.ds(j * 16, 16)] = jnp.full((16,), 1.0, dtype=_F32)

        @pl.when(s == 0)
        def _():
            pltpu.sync_copy(zn_hbm, deg_sh)

        plsc.subcore_barrier()

        nch = base + jnp.where(w < extra, 1, 0)

        def chunk(i, carry):
            off = (i * _NW + w) * _CH
            pltpu.sync_copy(dst_hbm.at[pl.ds(off, _CH)], dst_v)
            pltpu.sync_copy(ones_v, deg_sh.at[dst_v], add=True)
            return carry

        lax.fori_loop(0, nch, chunk, 0)

        plsc.subcore_barrier()

        @pl.when(s == 0)
        def _():
            pltpu.sync_copy(deg_sh, out_hbm.at[c])

    return deg


def _mm_body(x_ref, w_ref, o_ref):
    o_ref[...] = jnp.dot(x_ref[...], w_ref[...],
                         preferred_element_type=_F32)


def _tc_mm(x, w):
    n, d = x.shape
    h = w.shape[1]
    return pl.pallas_call(
        _mm_body,
        grid=(n // _ROWS,),
        in_specs=[pl.BlockSpec((_ROWS, d), lambda i: (i, 0)),
                  pl.BlockSpec((d, h), lambda i: (0, 0))],
        out_specs=pl.BlockSpec((_ROWS, h), lambda i: (i, 0)),
        out_shape=jax.ShapeDtypeStruct((n, h), _F32),
    )(x, w)


def _mid1_body(s_ref, dg_ref, x_ref, wr_ref, b_ref, wl_ref,
               h_ref, hl_ref, inv_ref):
    deg = dg_ref[0] + dg_ref[1]
    inv = 1.0 / jnp.maximum(deg, 1.0)
    agg = (s_ref[0] + s_ref[1]) * inv
    h = jnp.maximum(
        agg + jnp.dot(x_ref[...], wr_ref[...], preferred_element_type=_F32)
        + b_ref[...], 0.0)
    h_ref[...] = h
    hl_ref[...] = jnp.dot(h, wl_ref[...], preferred_element_type=_F32)
    inv_ref[...] = inv


def _tc_mid1(S, dg, x, wr, b, wl):
    n, d = x.shape
    h2 = wl.shape[1]
    return pl.pallas_call(
        _mid1_body,
        grid=(n // _ROWS,),
        in_specs=[pl.BlockSpec((2, _ROWS, d), lambda i: (0, i, 0)),
                  pl.BlockSpec((2, _ROWS, 1), lambda i: (0, i, 0)),
                  pl.BlockSpec((_ROWS, d), lambda i: (i, 0)),
                  pl.BlockSpec((d, d), lambda i: (0, 0)),
                  pl.BlockSpec((1, d), lambda i: (0, 0)),
                  pl.BlockSpec((d, h2), lambda i: (0, 0))],
        out_specs=[pl.BlockSpec((_ROWS, d), lambda i: (i, 0)),
                   pl.BlockSpec((_ROWS, h2), lambda i: (i, 0)),
                   pl.BlockSpec((_ROWS, 1), lambda i: (i, 0))],
        out_shape=[jax.ShapeDtypeStruct((n, d), _F32),
                   jax.ShapeDtypeStruct((n, h2), _F32),
                   jax.ShapeDtypeStruct((n, 1), _F32)],
    )(S, dg, x, wr, b, wl)


def _mid2_body(s_ref, inv_ref, x_ref, wr_ref, b_ref, h_ref):
    agg = (s_ref[0] + s_ref[1]) * inv_ref[...]
    h_ref[...] = jnp.maximum(
        agg + jnp.dot(x_ref[...], wr_ref[...], preferred_element_type=_F32)
        + b_ref[...], 0.0)


def _tc_mid2(S, inv, x, wr, b):
    n, d = x.shape
    return pl.pallas_call(
        _mid2_body,
        grid=(n // _ROWS,),
        in_specs=[pl.BlockSpec((2, _ROWS, d), lambda i: (0, i, 0)),
                  pl.BlockSpec((_ROWS, 1), lambda i: (i, 0)),
                  pl.BlockSpec((_ROWS, d), lambda i: (i, 0)),
                  pl.BlockSpec((d, d), lambda i: (0, 0)),
                  pl.BlockSpec((1, d), lambda i: (0, 0))],
        out_specs=pl.BlockSpec((_ROWS, d), lambda i: (i, 0)),
        out_shape=jax.ShapeDtypeStruct((n, d), _F32),
    )(S, inv, x, wr, b)


def _fin_body(s_ref, inv_ref, x_ref, wl_ref, wr_ref, b_ref, o_ref):
    agg = (s_ref[0] + s_ref[1]) * inv_ref[...]
    o_ref[...] = jnp.maximum(
        jnp.dot(agg, wl_ref[...], preferred_element_type=_F32)
        + jnp.dot(x_ref[...], wr_ref[...], preferred_element_type=_F32)
        + b_ref[...], 0.0)


def _tc_fin(S, inv, x, wl, wr, b):
    n, d = x.shape
    h = wr.shape[1]
    return pl.pallas_call(
        _fin_body,
        grid=(n // _ROWS,),
        in_specs=[pl.BlockSpec((2, _ROWS, d), lambda i: (0, i, 0)),
                  pl.BlockSpec((_ROWS, 1), lambda i: (i, 0)),
                  pl.BlockSpec((_ROWS, d), lambda i: (i, 0)),
                  pl.BlockSpec((d, h), lambda i: (0, 0)),
                  pl.BlockSpec((d, h), lambda i: (0, 0)),
                  pl.BlockSpec((1, h), lambda i: (0, 0))],
        out_specs=pl.BlockSpec((_ROWS, h), lambda i: (i, 0)),
        out_shape=jax.ShapeDtypeStruct((n, h), _F32),
    )(S, inv, x, wl, wr, b)


_sc_agg128 = _make_sc_agg(128)
_sc_deg = _make_sc_deg()


def kernel(x, edge_index, Wl1, Wr1, b1, Wl2, Wr2, b2, Wl3, Wr3, b3):
    src = edge_index[0]
    dst = edge_index[1]
    z128 = jnp.zeros((_N, 128), _F32)
    zn = jnp.zeros((_N,), _F32)

    dg = _sc_deg(dst, zn)
    hl1 = _tc_mm(x, Wl1)
    S1 = _sc_agg128(hl1, src, dst, z128)
    h2, hl2, inv = _tc_mid1(S1, dg.reshape(2, _N, 1), x,
                            Wr1, b1.reshape(1, 128), Wl2)
    S2 = _sc_agg128(hl2, src, dst, z128)
    h3 = _tc_mid2(S2, inv, h2, Wr2, b2.reshape(1, 128))
    S3 = _sc_agg128(h3, src, dst, z128)
    out = _tc_fin(S3, inv, h3, Wl3, Wr3, b3.reshape(1, 64))
    return out
